# Initial kernel scaffold; baseline (speedup 1.0000x reference)
import jax
import jax.numpy as jnp
from jax.experimental import pallas as pl


def kernel(state_vector, probabilities):
    raise NotImplementedError("write your pallas kernel here")


# speedup vs baseline: 1.0043x; 1.0043x over previous
import jax
import jax.numpy as jnp
from jax.experimental import pallas as pl
from jax.experimental.pallas import tpu as pltpu

_B = 64
_N = 100000
_BLK = 4096
_NB = (_N + _BLK - 1) // _BLK


def _reduce_body(p_ref, max_ref, idx_ref):
    j = pl.program_id(0)

    @pl.when(j == 0)
    def _init():
        max_ref[...] = jnp.full((_B,), -jnp.inf, jnp.float32)
        idx_ref[...] = jnp.zeros((_B,), jnp.int32)

    x = p_ref[...]
    col = jax.lax.broadcasted_iota(jnp.int32, (_B, _BLK), 1) + j * _BLK
    x = jnp.where(col < _N, x, -jnp.inf)
    blk_max = jnp.max(x, axis=1)
    is_max = x == blk_max[:, None]
    blk_idx = jnp.min(jnp.where(is_max, col, _N), axis=1)

    better = blk_max > max_ref[...]
    max_ref[...] = jnp.where(better, blk_max, max_ref[...])
    idx_ref[...] = jnp.where(better, blk_idx, idx_ref[...])


def _onehot_body(idx_ref, out_ref):
    j = pl.program_id(0)
    col = jax.lax.broadcasted_iota(jnp.int32, (_B, _BLK), 1) + j * _BLK
    out_ref[...] = (col == idx_ref[...][:, None]).astype(jnp.float32)


def kernel(state_vector, probabilities):
    del state_vector

    max_val, arg_idx = pl.pallas_call(
        _reduce_body,
        grid=(_NB,),
        in_specs=[pl.BlockSpec((_B, _BLK), lambda j: (0, j))],
        out_specs=[
            pl.BlockSpec((_B,), lambda j: (0,)),
            pl.BlockSpec((_B,), lambda j: (0,)),
        ],
        out_shape=[
            jax.ShapeDtypeStruct((_B,), jnp.float32),
            jax.ShapeDtypeStruct((_B,), jnp.int32),
        ],
    )(probabilities)

    collapsed = pl.pallas_call(
        _onehot_body,
        grid=(_NB,),
        in_specs=[pl.BlockSpec((_B,), lambda j: (0,))],
        out_specs=pl.BlockSpec((_B, _BLK), lambda j: (0, j)),
        out_shape=jax.ShapeDtypeStruct((_B, _N), jnp.float32),
    )(arg_idx)

    return collapsed, max_val
